# baseline (speedup 1.0000x reference)
import jax
import jax.numpy as jnp
from jax.experimental import pallas as pl
from jax.experimental.pallas import tpu as pltpu


def _row_copy(s_ref, t_ref, o_ref):
    s = s_ref[0]
    o_ref[...] = t_ref[pl.ds(s, 1), :]


def kernel(sensor, table):
    s = jnp.reshape(jnp.asarray(sensor, jnp.int32), (1,))
    out = pl.pallas_call(
        _row_copy,
        in_specs=[
            pl.BlockSpec(memory_space=pltpu.SMEM),
            pl.BlockSpec(memory_space=pltpu.VMEM),
        ],
        out_shape=jax.ShapeDtypeStruct((1, table.shape[1]), table.dtype),
    )(s, table)
    return out[0]


# speedup vs baseline: 1.1058x; 1.1058x over previous
import jax
import jax.numpy as jnp
from jax.experimental import pallas as pl
from jax.experimental.pallas import tpu as pltpu


def _row_dma(s_ref, t_ref, o_ref, sem):
    s = s_ref[0]
    pltpu.make_async_copy(t_ref.at[pl.ds(s, 1), :], o_ref, sem).start()
    pltpu.make_async_copy(t_ref.at[pl.ds(s, 1), :], o_ref, sem).wait()


def kernel(sensor, table):
    s = jnp.reshape(jnp.asarray(sensor, jnp.int32), (1,))
    out = pl.pallas_call(
        _row_dma,
        in_specs=[
            pl.BlockSpec(memory_space=pltpu.SMEM),
            pl.BlockSpec(memory_space=pl.ANY),
        ],
        out_shape=jax.ShapeDtypeStruct((1, table.shape[1]), table.dtype),
        scratch_shapes=[pltpu.SemaphoreType.DMA],
    )(s, table)
    return out[0]
